# Initial kernel scaffold; baseline (speedup 1.0000x reference)
import jax
import jax.numpy as jnp
from jax.experimental import pallas as pl


def kernel(lig_x, lig_edge_index, lig_edge_attr, rec_feat, rec_edge_index, rec_edge_attr, params):
    raise NotImplementedError("write your pallas kernel here")


# speedup vs baseline: 1.3782x; 1.3782x over previous
import jax
import jax.numpy as jnp
from jax import lax
from jax.experimental import pallas as pl
from jax.experimental.pallas import tpu as pltpu
from jax.experimental.pallas import tpu_sc as plsc

_G, _NPG, _N, _E = 10, 1000, 10000, 160000
_DH = 32
_NC, _NS = 2, 16
_NW = _NC * _NS
_RPT = 128
_NE2 = 2 * _E
_NROWS = _NE2 // _RPT
_N2 = 2 * _N


def _proj_body(lx_ref, feat_ref, emb_ref, lw_ref, lb_ref, rw_ref, rb_ref, out_ref):
    h_lig = jax.nn.relu(
        jnp.dot(lx_ref[...], lw_ref[...], preferred_element_type=jnp.float32)
        + lb_ref[...])
    emb_proj = jnp.dot(emb_ref[...], rw_ref[...], preferred_element_type=jnp.float32)
    feat = feat_ref[...]
    onehot = (lax.broadcasted_iota(jnp.int32, (feat.shape[0], 32), 1)
              == feat).astype(jnp.float32)
    h_rec = jax.nn.relu(
        jnp.dot(onehot, emb_proj, preferred_element_type=jnp.float32) + rb_ref[...])
    out_ref[0] = h_lig
    out_ref[1] = h_rec


def _run_proj(lig_x, feat, emb_pad, lw, lb, rw, rb):
    blk = 2000
    nb = _N // blk
    return pl.pallas_call(
        _proj_body,
        grid=(nb,),
        in_specs=[
            pl.BlockSpec((blk, 128), lambda i: (i, 0)),
            pl.BlockSpec((blk, 1), lambda i: (i, 0)),
            pl.BlockSpec((32, 64), lambda i: (0, 0)),
            pl.BlockSpec((128, 32), lambda i: (0, 0)),
            pl.BlockSpec((1, 32), lambda i: (0, 0)),
            pl.BlockSpec((64, 32), lambda i: (0, 0)),
            pl.BlockSpec((1, 32), lambda i: (0, 0)),
        ],
        out_specs=pl.BlockSpec((2, blk, 32), lambda i: (0, i, 0)),
        out_shape=jax.ShapeDtypeStruct((2, _N, 32), jnp.float32),
    )(lig_x, feat, emb_pad, lw, lb, rw, rb)


def _gather_body(tbl_hbm, idx_hbm, out_hbm, idx_v, rows_v, sem):
    wid = lax.axis_index("s") * _NC + lax.axis_index("c")
    n_extra = _NROWS - (_NROWS // _NW) * _NW
    n_it = jnp.where(wid < n_extra, _NROWS // _NW + 1, _NROWS // _NW)

    def step(t, _):
        j = wid + t * _NW
        pltpu.sync_copy(idx_hbm.at[j], idx_v)
        pltpu.async_copy(tbl_hbm.at[idx_v], rows_v, sem).wait()
        pltpu.sync_copy(rows_v, out_hbm.at[pl.ds(j * _RPT, _RPT)])
        return _

    lax.fori_loop(0, n_it, step, 0)


def _run_gather(tbl, idx2d):
    mesh = plsc.VectorSubcoreMesh(core_axis_name="c", subcore_axis_name="s")
    f = pl.kernel(
        _gather_body,
        out_type=jax.ShapeDtypeStruct((_NE2, 32), jnp.float32),
        mesh=mesh,
        scratch_types=[
            pltpu.VMEM((_RPT,), jnp.int32),
            pltpu.VMEM((_RPT, 32), jnp.float32),
            pltpu.SemaphoreType.DMA,
        ],
        compiler_params=pltpu.CompilerParams(use_tc_tiling_on_sc=False),
    )
    return f(tbl, idx2d)


def _msg_body(ea_ref, hs_ref, w1_ref, b1_ref, w2_ref, b2_ref, out_ref):
    u = jax.nn.relu(
        jnp.dot(ea_ref[0], w1_ref[0], preferred_element_type=jnp.float32)
        + b1_ref[0])
    w = jnp.dot(u, w2_ref[0], preferred_element_type=jnp.float32) + b2_ref[0]
    hs = hs_ref[0]
    blk = hs.shape[0]
    acc = jnp.zeros((blk, 128), jnp.float32)
    for grp in range(8):
        hb = jnp.concatenate(
            [jnp.broadcast_to(hs[:, 4 * grp + t:4 * grp + t + 1], (blk, 32))
             for t in range(4)], axis=1)
        acc = acc + w[:, 128 * grp:128 * (grp + 1)] * hb
    out_ref[0] = (acc[:, 0:32] + acc[:, 32:64] + acc[:, 64:96]
                  + acc[:, 96:128])


def _run_msg(ea_s, hs_s, w1_s, b1_s, w2_s, b2_s):
    blk = 2000
    nb = _E // blk
    return pl.pallas_call(
        _msg_body,
        grid=(2, nb),
        in_specs=[
            pl.BlockSpec((1, blk, 16), lambda g, e: (g, e, 0)),
            pl.BlockSpec((1, blk, 32), lambda g, e: (g, e, 0)),
            pl.BlockSpec((1, 16, 128), lambda g, e: (g, 0, 0)),
            pl.BlockSpec((1, 1, 128), lambda g, e: (g, 0, 0)),
            pl.BlockSpec((1, 128, 1024), lambda g, e: (g, 0, 0)),
            pl.BlockSpec((1, 1, 1024), lambda g, e: (g, 0, 0)),
        ],
        out_specs=pl.BlockSpec((1, blk, 32), lambda g, e: (g, e, 0)),
        out_shape=jax.ShapeDtypeStruct((2, _E, 32), jnp.float32),
    )(ea_s, hs_s, w1_s, b1_s, w2_s, b2_s)


def _scatter_body(msg_hbm, dst_hbm, zero_hbm, out_hbm, acc_sh, idx_v, rows_v):
    c = lax.axis_index("c")
    s = lax.axis_index("s")
    wid = s * _NC + c

    @pl.when(s == 0)
    def _():
        pltpu.sync_copy(zero_hbm, acc_sh)

    plsc.subcore_barrier()

    n_extra = _NROWS - (_NROWS // _NW) * _NW
    n_it = jnp.where(wid < n_extra, _NROWS // _NW + 1, _NROWS // _NW)

    def step(t, _):
        j = wid + t * _NW
        pltpu.sync_copy(dst_hbm.at[j], idx_v)
        pltpu.sync_copy(msg_hbm.at[pl.ds(j * _RPT, _RPT)], rows_v)
        pltpu.sync_copy(rows_v, acc_sh.at[idx_v], add=True)
        return _

    lax.fori_loop(0, n_it, step, 0)
    plsc.subcore_barrier()

    rows = _N2 // _NS
    pltpu.sync_copy(acc_sh.at[pl.ds(s * rows, rows)],
                    out_hbm.at[c, pl.ds(s * rows, rows)])


def _run_scatter(msg_cat, dst2d, zeros):
    mesh = plsc.VectorSubcoreMesh(core_axis_name="c", subcore_axis_name="s")
    f = pl.kernel(
        _scatter_body,
        out_type=jax.ShapeDtypeStruct((_NC, _N2, 32), jnp.float32),
        mesh=mesh,
        scratch_types=[
            pltpu.VMEM_SHARED((_N2, 32), jnp.float32),
            pltpu.VMEM((_RPT,), jnp.int32),
            pltpu.VMEM((_RPT, 32), jnp.float32),
        ],
        compiler_params=pltpu.CompilerParams(use_tc_tiling_on_sc=False),
    )
    return f(msg_cat, dst2d, zeros)


def _gru_body(parts_ref, nnb_ref, h_ref, wi_ref, bi_ref, wh_ref, bh_ref, out_ref):
    agg = parts_ref[0, 0] + parts_ref[1, 0] + nnb_ref[0]
    m = jax.nn.relu(agg)
    h = h_ref[0]
    gi = jnp.dot(m, wi_ref[0], preferred_element_type=jnp.float32) + bi_ref[0]
    gh = jnp.dot(h, wh_ref[0], preferred_element_type=jnp.float32) + bh_ref[0]
    r = jax.nn.sigmoid(gi[:, 0:32] + gh[:, 0:32])
    z = jax.nn.sigmoid(gi[:, 32:64] + gh[:, 32:64])
    n = jnp.tanh(gi[:, 64:96] + r * gh[:, 64:96])
    out_ref[0] = (1.0 - z) * n + z * h


def _run_gru(parts, nnb_s, h_s, wi_s, bi_s, wh_s, bh_s):
    blk = 2000
    nb = _N // blk
    return pl.pallas_call(
        _gru_body,
        grid=(2, nb),
        in_specs=[
            pl.BlockSpec((2, 1, blk, 32), lambda g, b: (0, g, b, 0)),
            pl.BlockSpec((1, 1, 32), lambda g, b: (g, 0, 0)),
            pl.BlockSpec((1, blk, 32), lambda g, b: (g, b, 0)),
            pl.BlockSpec((1, 32, 96), lambda g, b: (g, 0, 0)),
            pl.BlockSpec((1, 1, 96), lambda g, b: (g, 0, 0)),
            pl.BlockSpec((1, 32, 96), lambda g, b: (g, 0, 0)),
            pl.BlockSpec((1, 1, 96), lambda g, b: (g, 0, 0)),
        ],
        out_specs=pl.BlockSpec((1, blk, 32), lambda g, b: (g, b, 0)),
        out_shape=jax.ShapeDtypeStruct((2, _N, 32), jnp.float32),
    )(parts, nnb_s, h_s, wi_s, bi_s, wh_s, bh_s)


def _atn_body(hid_ref, wq_ref, bq_ref, wk_ref, bk_ref, wv_ref, bv_ref,
              wo_ref, bo_ref, cw_ref, cb_ref, rw_ref, rb_ref, lw_ref, lb_ref,
              out_ref):
    lig = hid_ref[0]
    rec = hid_ref[1]
    q = jnp.dot(lig, wq_ref[...], preferred_element_type=jnp.float32) + bq_ref[...]
    k = jnp.dot(rec, wk_ref[...], preferred_element_type=jnp.float32) + bk_ref[...]
    v = jnp.dot(rec, wv_ref[...], preferred_element_type=jnp.float32) + bv_ref[...]
    scores = lax.dot_general(q, k, (((1,), (1,)), ((), ())),
                             preferred_element_type=jnp.float32) * (1.0 / (_DH ** 0.5))
    mx = jnp.max(scores, axis=1, keepdims=True)
    ex = jnp.exp(scores - mx)
    a = ex / jnp.sum(ex, axis=1, keepdims=True)
    av = jnp.dot(a, v, preferred_element_type=jnp.float32)
    atn = jnp.dot(av, wo_ref[...], preferred_element_type=jnp.float32) + bo_ref[...]
    cat = jnp.concatenate([lig, atn], axis=1)
    lcomb = jnp.dot(cat, cw_ref[...], preferred_element_type=jnp.float32) + cb_ref[...]
    wr = jax.nn.sigmoid(
        jnp.dot(rec, rw_ref[...], preferred_element_type=jnp.float32) + rb_ref[...])
    hs_rec = jnp.sum(wr * rec, axis=0, keepdims=True)
    hm_rec = jnp.max(rec, axis=0, keepdims=True)
    wl = jax.nn.sigmoid(
        jnp.dot(lcomb, lw_ref[...], preferred_element_type=jnp.float32) + lb_ref[...])
    hs_lig = jnp.sum(wl * lcomb, axis=0, keepdims=True)
    hm_lig = jnp.max(lcomb, axis=0, keepdims=True)
    out_ref[0] = jnp.concatenate([hs_rec, hm_rec, hs_lig, hm_lig], axis=1)


def _run_atn(hid_s, wqt, bq, wkt, bk, wvt, bv, wot, bo, cw, cb, rw, rb, lw, lb):
    small = lambda shape: pl.BlockSpec(shape, lambda g: tuple(0 for _ in shape))
    return pl.pallas_call(
        _atn_body,
        grid=(_G,),
        in_specs=[
            pl.BlockSpec((2, _NPG, 32), lambda g: (0, g, 0)),
            small((32, 32)), small((1, 32)),
            small((32, 32)), small((1, 32)),
            small((32, 32)), small((1, 32)),
            small((32, 32)), small((1, 32)),
            small((64, 32)), small((1, 32)),
            small((32, 1)), small((1, 1)),
            small((32, 1)), small((1, 1)),
        ],
        out_specs=pl.BlockSpec((1, 1, 128), lambda g: (g, 0, 0)),
        out_shape=jax.ShapeDtypeStruct((_G, 1, 128), jnp.float32),
    )(hid_s, wqt, bq, wkt, bk, wvt, bv, wot, bo, cw, cb, rw, rb, lw, lb)


def _mlp_body(x_ref, w1_ref, b1_ref, w2_ref, b2_ref, wo_ref, bo_ref, out_ref):
    x = jnp.dot(x_ref[...], w1_ref[...], preferred_element_type=jnp.float32) + b1_ref[...]
    x = jnp.where(x > 0, x, 0.01 * x)
    x = jnp.dot(x, w2_ref[...], preferred_element_type=jnp.float32) + b2_ref[...]
    x = jnp.where(x > 0, x, 0.01 * x)
    out_ref[...] = jnp.dot(x, wo_ref[...], preferred_element_type=jnp.float32) + bo_ref[...]


def _run_mlp(x, w1, b1, w2, b2, wo, bo):
    return pl.pallas_call(
        _mlp_body,
        out_shape=jax.ShapeDtypeStruct((_G, 1), jnp.float32),
    )(x, w1, b1, w2, b2, wo, bo)


def kernel(lig_x, lig_edge_index, lig_edge_attr, rec_feat, rec_edge_index,
           rec_edge_attr, params):
    pg, pr = params['lig_gnn'], params['rec_gnn']
    f32 = jnp.float32

    emb_pad = jnp.zeros((32, 64), f32).at[:21].set(params['rec_embed'])
    h_s = _run_proj(lig_x, rec_feat, emb_pad,
                    pg['proj_W'], pg['proj_b'].reshape(1, 32),
                    pr['proj_W'], pr['proj_b'].reshape(1, 32))

    tbl = h_s.reshape(_N2, 32)
    src_cat = jnp.concatenate(
        [lig_edge_index[0], rec_edge_index[0] + _N]).reshape(_NROWS, _RPT)
    dst_cat = jnp.concatenate(
        [lig_edge_index[1], rec_edge_index[1] + _N]).reshape(_NROWS, _RPT)

    h_src = _run_gather(tbl, src_cat).reshape(2, _E, 32)

    ea_s = jnp.stack([lig_edge_attr, rec_edge_attr])
    w1_s = jnp.stack([pg['eW1'], pr['eW1']])
    b1_s = jnp.stack([pg['eb1'], pr['eb1']]).reshape(2, 1, 128)
    w2_s = jnp.stack([pg['eW2'], pr['eW2']])
    b2_s = jnp.stack([pg['eb2'], pr['eb2']]).reshape(2, 1, 1024)
    msg = _run_msg(ea_s, h_src, w1_s, b1_s, w2_s, b2_s)

    parts = _run_scatter(msg.reshape(_NE2, 32), dst_cat,
                         jnp.zeros((_N2, 32), f32)).reshape(2, 2, _N, 32)

    nnb_s = jnp.stack([pg['nn_bias'], pr['nn_bias']]).reshape(2, 1, 32)
    wi_s = jnp.stack([pg['gru_Wi'], pr['gru_Wi']])
    bi_s = jnp.stack([pg['gru_bi'], pr['gru_bi']]).reshape(2, 1, 96)
    wh_s = jnp.stack([pg['gru_Wh'], pr['gru_Wh']])
    bh_s = jnp.stack([pg['gru_bh'], pr['gru_bh']]).reshape(2, 1, 96)
    hid_s = _run_gru(parts, nnb_s, h_s, wi_s, bi_s, wh_s, bh_s)

    a = params['atn']
    feats = _run_atn(
        hid_s,
        a['Wq'].T, a['bq'].reshape(1, 32), a['Wk'].T, a['bk'].reshape(1, 32),
        a['Wv'].T, a['bv'].reshape(1, 32), a['Wo'].T, a['bo'].reshape(1, 32),
        params['comb_W'], params['comb_b'].reshape(1, 32),
        params['rec_ro_W'], params['rec_ro_b'].reshape(1, 1),
        params['lig_ro_W'], params['lig_ro_b'].reshape(1, 1),
    )

    m = params['mlp']
    return _run_mlp(feats.reshape(_G, 128),
                    m['W1'], m['b1'].reshape(1, 256),
                    m['W2'], m['b2'].reshape(1, 128),
                    m['Wo'], m['bo'].reshape(1, 1))


# speedup vs baseline: 1.3791x; 1.0007x over previous
import jax
import jax.numpy as jnp
from jax import lax
from jax.experimental import pallas as pl
from jax.experimental.pallas import tpu as pltpu
from jax.experimental.pallas import tpu_sc as plsc

_G, _NPG, _N, _E = 10, 1000, 10000, 160000
_DH = 32
_NC, _NS = 2, 16
_NW = _NC * _NS
_RPT = 128
_NE2 = 2 * _E
_NROWS = _NE2 // _RPT
_N2 = 2 * _N


def _proj_body(lx_ref, feat_ref, emb_ref, lw_ref, lb_ref, rw_ref, rb_ref, out_ref):
    h_lig = jax.nn.relu(
        jnp.dot(lx_ref[...], lw_ref[...], preferred_element_type=jnp.float32)
        + lb_ref[...])
    emb_proj = jnp.dot(emb_ref[...], rw_ref[...], preferred_element_type=jnp.float32)
    feat = feat_ref[...]
    onehot = (lax.broadcasted_iota(jnp.int32, (feat.shape[0], 32), 1)
              == feat).astype(jnp.float32)
    h_rec = jax.nn.relu(
        jnp.dot(onehot, emb_proj, preferred_element_type=jnp.float32) + rb_ref[...])
    out_ref[0] = h_lig
    out_ref[1] = h_rec


def _run_proj(lig_x, feat, emb_pad, lw, lb, rw, rb):
    blk = 2000
    nb = _N // blk
    return pl.pallas_call(
        _proj_body,
        grid=(nb,),
        in_specs=[
            pl.BlockSpec((blk, 128), lambda i: (i, 0)),
            pl.BlockSpec((blk, 1), lambda i: (i, 0)),
            pl.BlockSpec((32, 64), lambda i: (0, 0)),
            pl.BlockSpec((128, 32), lambda i: (0, 0)),
            pl.BlockSpec((1, 32), lambda i: (0, 0)),
            pl.BlockSpec((64, 32), lambda i: (0, 0)),
            pl.BlockSpec((1, 32), lambda i: (0, 0)),
        ],
        out_specs=pl.BlockSpec((2, blk, 32), lambda i: (0, i, 0)),
        out_shape=jax.ShapeDtypeStruct((2, _N, 32), jnp.float32),
    )(lig_x, feat, emb_pad, lw, lb, rw, rb)


def _gather_body(tbl_hbm, idx_hbm, out_hbm, idx_v, rows_v, sem):
    wid = lax.axis_index("s") * _NC + lax.axis_index("c")
    n_extra = _NROWS - (_NROWS // _NW) * _NW
    n_it = jnp.where(wid < n_extra, _NROWS // _NW + 1, _NROWS // _NW)

    def step(t, _):
        j = wid + t * _NW
        pltpu.sync_copy(idx_hbm.at[j], idx_v)
        pltpu.async_copy(tbl_hbm.at[idx_v], rows_v, sem).wait()
        pltpu.sync_copy(rows_v, out_hbm.at[pl.ds(j * _RPT, _RPT)])
        return _

    lax.fori_loop(0, n_it, step, 0)


def _run_gather(tbl, idx2d):
    mesh = plsc.VectorSubcoreMesh(core_axis_name="c", subcore_axis_name="s")
    f = pl.kernel(
        _gather_body,
        out_type=jax.ShapeDtypeStruct((_NE2, 32), jnp.float32),
        mesh=mesh,
        scratch_types=[
            pltpu.VMEM((_RPT,), jnp.int32),
            pltpu.VMEM((_RPT, 32), jnp.float32),
            pltpu.SemaphoreType.DMA,
        ],
        compiler_params=pltpu.CompilerParams(use_tc_tiling_on_sc=False),
    )
    return f(tbl, idx2d)


def _msg_body(ea_ref, hs_ref, w1_ref, b1_ref, w2_ref, b2_ref, out_ref):
    u = jax.nn.relu(
        jnp.dot(ea_ref[0], w1_ref[0], preferred_element_type=jnp.float32)
        + b1_ref[0])
    w = jnp.dot(u.astype(jnp.bfloat16), w2_ref[0],
                preferred_element_type=jnp.float32) + b2_ref[0]
    hs = hs_ref[0]
    blk = hs.shape[0]
    acc = jnp.zeros((blk, 128), jnp.float32)
    for grp in range(8):
        hb = jnp.concatenate(
            [jnp.broadcast_to(hs[:, 4 * grp + t:4 * grp + t + 1], (blk, 32))
             for t in range(4)], axis=1)
        acc = acc + w[:, 128 * grp:128 * (grp + 1)] * hb
    out_ref[0] = (acc[:, 0:32] + acc[:, 32:64] + acc[:, 64:96]
                  + acc[:, 96:128])


def _run_msg(ea_s, hs_s, w1_s, b1_s, w2_s, b2_s):
    blk = 2000
    nb = _E // blk
    return pl.pallas_call(
        _msg_body,
        grid=(2, nb),
        in_specs=[
            pl.BlockSpec((1, blk, 16), lambda g, e: (g, e, 0)),
            pl.BlockSpec((1, blk, 32), lambda g, e: (g, e, 0)),
            pl.BlockSpec((1, 16, 128), lambda g, e: (g, 0, 0)),
            pl.BlockSpec((1, 1, 128), lambda g, e: (g, 0, 0)),
            pl.BlockSpec((1, 128, 1024), lambda g, e: (g, 0, 0)),
            pl.BlockSpec((1, 1, 1024), lambda g, e: (g, 0, 0)),
        ],
        compiler_params=pltpu.CompilerParams(
            dimension_semantics=("arbitrary", "arbitrary")),
        out_specs=pl.BlockSpec((1, blk, 32), lambda g, e: (g, e, 0)),
        out_shape=jax.ShapeDtypeStruct((2, _E, 32), jnp.float32),
    )(ea_s, hs_s, w1_s, b1_s, w2_s, b2_s)


def _scatter_body(msg_hbm, dst_hbm, zero_hbm, out_hbm, acc_sh, idx_v, rows_v):
    c = lax.axis_index("c")
    s = lax.axis_index("s")
    wid = s * _NC + c

    @pl.when(s == 0)
    def _():
        pltpu.sync_copy(zero_hbm, acc_sh)

    plsc.subcore_barrier()

    n_extra = _NROWS - (_NROWS // _NW) * _NW
    n_it = jnp.where(wid < n_extra, _NROWS // _NW + 1, _NROWS // _NW)

    def step(t, _):
        j = wid + t * _NW
        pltpu.sync_copy(dst_hbm.at[j], idx_v)
        pltpu.sync_copy(msg_hbm.at[pl.ds(j * _RPT, _RPT)], rows_v)
        pltpu.sync_copy(rows_v, acc_sh.at[idx_v], add=True)
        return _

    lax.fori_loop(0, n_it, step, 0)
    plsc.subcore_barrier()

    rows = _N2 // _NS
    pltpu.sync_copy(acc_sh.at[pl.ds(s * rows, rows)],
                    out_hbm.at[c, pl.ds(s * rows, rows)])


def _run_scatter(msg_cat, dst2d, zeros):
    mesh = plsc.VectorSubcoreMesh(core_axis_name="c", subcore_axis_name="s")
    f = pl.kernel(
        _scatter_body,
        out_type=jax.ShapeDtypeStruct((_NC, _N2, 32), jnp.float32),
        mesh=mesh,
        scratch_types=[
            pltpu.VMEM_SHARED((_N2, 32), jnp.float32),
            pltpu.VMEM((_RPT,), jnp.int32),
            pltpu.VMEM((_RPT, 32), jnp.float32),
        ],
        compiler_params=pltpu.CompilerParams(use_tc_tiling_on_sc=False),
    )
    return f(msg_cat, dst2d, zeros)


def _gru_body(parts_ref, nnb_ref, h_ref, wi_ref, bi_ref, wh_ref, bh_ref, out_ref):
    agg = parts_ref[0, 0] + parts_ref[1, 0] + nnb_ref[0]
    m = jax.nn.relu(agg)
    h = h_ref[0]
    gi = jnp.dot(m, wi_ref[0], preferred_element_type=jnp.float32) + bi_ref[0]
    gh = jnp.dot(h, wh_ref[0], preferred_element_type=jnp.float32) + bh_ref[0]
    r = jax.nn.sigmoid(gi[:, 0:32] + gh[:, 0:32])
    z = jax.nn.sigmoid(gi[:, 32:64] + gh[:, 32:64])
    n = jnp.tanh(gi[:, 64:96] + r * gh[:, 64:96])
    out_ref[0] = (1.0 - z) * n + z * h


def _run_gru(parts, nnb_s, h_s, wi_s, bi_s, wh_s, bh_s):
    blk = 2000
    nb = _N // blk
    return pl.pallas_call(
        _gru_body,
        grid=(2, nb),
        in_specs=[
            pl.BlockSpec((2, 1, blk, 32), lambda g, b: (0, g, b, 0)),
            pl.BlockSpec((1, 1, 32), lambda g, b: (g, 0, 0)),
            pl.BlockSpec((1, blk, 32), lambda g, b: (g, b, 0)),
            pl.BlockSpec((1, 32, 96), lambda g, b: (g, 0, 0)),
            pl.BlockSpec((1, 1, 96), lambda g, b: (g, 0, 0)),
            pl.BlockSpec((1, 32, 96), lambda g, b: (g, 0, 0)),
            pl.BlockSpec((1, 1, 96), lambda g, b: (g, 0, 0)),
        ],
        out_specs=pl.BlockSpec((1, blk, 32), lambda g, b: (g, b, 0)),
        out_shape=jax.ShapeDtypeStruct((2, _N, 32), jnp.float32),
    )(parts, nnb_s, h_s, wi_s, bi_s, wh_s, bh_s)


def _atn_body(hid_ref, wq_ref, bq_ref, wk_ref, bk_ref, wv_ref, bv_ref,
              wo_ref, bo_ref, cw_ref, cb_ref, rw_ref, rb_ref, lw_ref, lb_ref,
              out_ref):
    lig = hid_ref[0]
    rec = hid_ref[1]
    q = jnp.dot(lig, wq_ref[...], preferred_element_type=jnp.float32) + bq_ref[...]
    k = jnp.dot(rec, wk_ref[...], preferred_element_type=jnp.float32) + bk_ref[...]
    v = jnp.dot(rec, wv_ref[...], preferred_element_type=jnp.float32) + bv_ref[...]
    scores = lax.dot_general(q, k, (((1,), (1,)), ((), ())),
                             preferred_element_type=jnp.float32) * (1.0 / (_DH ** 0.5))
    mx = jnp.max(scores, axis=1, keepdims=True)
    ex = jnp.exp(scores - mx)
    a = ex / jnp.sum(ex, axis=1, keepdims=True)
    av = jnp.dot(a, v, preferred_element_type=jnp.float32)
    atn = jnp.dot(av, wo_ref[...], preferred_element_type=jnp.float32) + bo_ref[...]
    cat = jnp.concatenate([lig, atn], axis=1)
    lcomb = jnp.dot(cat, cw_ref[...], preferred_element_type=jnp.float32) + cb_ref[...]
    wr = jax.nn.sigmoid(
        jnp.dot(rec, rw_ref[...], preferred_element_type=jnp.float32) + rb_ref[...])
    hs_rec = jnp.sum(wr * rec, axis=0, keepdims=True)
    hm_rec = jnp.max(rec, axis=0, keepdims=True)
    wl = jax.nn.sigmoid(
        jnp.dot(lcomb, lw_ref[...], preferred_element_type=jnp.float32) + lb_ref[...])
    hs_lig = jnp.sum(wl * lcomb, axis=0, keepdims=True)
    hm_lig = jnp.max(lcomb, axis=0, keepdims=True)
    out_ref[0] = jnp.concatenate([hs_rec, hm_rec, hs_lig, hm_lig], axis=1)


def _run_atn(hid_s, wqt, bq, wkt, bk, wvt, bv, wot, bo, cw, cb, rw, rb, lw, lb):
    small = lambda shape: pl.BlockSpec(shape, lambda g: tuple(0 for _ in shape))
    return pl.pallas_call(
        _atn_body,
        grid=(_G,),
        in_specs=[
            pl.BlockSpec((2, _NPG, 32), lambda g: (0, g, 0)),
            small((32, 32)), small((1, 32)),
            small((32, 32)), small((1, 32)),
            small((32, 32)), small((1, 32)),
            small((32, 32)), small((1, 32)),
            small((64, 32)), small((1, 32)),
            small((32, 1)), small((1, 1)),
            small((32, 1)), small((1, 1)),
        ],
        out_specs=pl.BlockSpec((1, 1, 128), lambda g: (g, 0, 0)),
        out_shape=jax.ShapeDtypeStruct((_G, 1, 128), jnp.float32),
    )(hid_s, wqt, bq, wkt, bk, wvt, bv, wot, bo, cw, cb, rw, rb, lw, lb)


def _mlp_body(x_ref, w1_ref, b1_ref, w2_ref, b2_ref, wo_ref, bo_ref, out_ref):
    x = jnp.dot(x_ref[...], w1_ref[...], preferred_element_type=jnp.float32) + b1_ref[...]
    x = jnp.where(x > 0, x, 0.01 * x)
    x = jnp.dot(x, w2_ref[...], preferred_element_type=jnp.float32) + b2_ref[...]
    x = jnp.where(x > 0, x, 0.01 * x)
    out_ref[...] = jnp.dot(x, wo_ref[...], preferred_element_type=jnp.float32) + bo_ref[...]


def _run_mlp(x, w1, b1, w2, b2, wo, bo):
    return pl.pallas_call(
        _mlp_body,
        out_shape=jax.ShapeDtypeStruct((_G, 1), jnp.float32),
    )(x, w1, b1, w2, b2, wo, bo)


def kernel(lig_x, lig_edge_index, lig_edge_attr, rec_feat, rec_edge_index,
           rec_edge_attr, params):
    pg, pr = params['lig_gnn'], params['rec_gnn']
    f32 = jnp.float32

    emb_pad = jnp.zeros((32, 64), f32).at[:21].set(params['rec_embed'])
    h_s = _run_proj(lig_x, rec_feat, emb_pad,
                    pg['proj_W'], pg['proj_b'].reshape(1, 32),
                    pr['proj_W'], pr['proj_b'].reshape(1, 32))

    tbl = h_s.reshape(_N2, 32)
    src_cat = jnp.concatenate(
        [lig_edge_index[0], rec_edge_index[0] + _N]).reshape(_NROWS, _RPT)
    dst_cat = jnp.concatenate(
        [lig_edge_index[1], rec_edge_index[1] + _N]).reshape(_NROWS, _RPT)

    h_src = _run_gather(tbl, src_cat).reshape(2, _E, 32)

    ea_s = jnp.stack([lig_edge_attr, rec_edge_attr])
    w1_s = jnp.stack([pg['eW1'], pr['eW1']])
    b1_s = jnp.stack([pg['eb1'], pr['eb1']]).reshape(2, 1, 128)
    w2_s = jnp.stack([pg['eW2'], pr['eW2']]).astype(jnp.bfloat16)
    b2_s = jnp.stack([pg['eb2'], pr['eb2']]).reshape(2, 1, 1024)
    msg = _run_msg(ea_s, h_src, w1_s, b1_s, w2_s, b2_s)

    parts = _run_scatter(msg.reshape(_NE2, 32), dst_cat,
                         jnp.zeros((_N2, 32), f32)).reshape(2, 2, _N, 32)

    nnb_s = jnp.stack([pg['nn_bias'], pr['nn_bias']]).reshape(2, 1, 32)
    wi_s = jnp.stack([pg['gru_Wi'], pr['gru_Wi']])
    bi_s = jnp.stack([pg['gru_bi'], pr['gru_bi']]).reshape(2, 1, 96)
    wh_s = jnp.stack([pg['gru_Wh'], pr['gru_Wh']])
    bh_s = jnp.stack([pg['gru_bh'], pr['gru_bh']]).reshape(2, 1, 96)
    hid_s = _run_gru(parts, nnb_s, h_s, wi_s, bi_s, wh_s, bh_s)

    a = params['atn']
    feats = _run_atn(
        hid_s,
        a['Wq'].T, a['bq'].reshape(1, 32), a['Wk'].T, a['bk'].reshape(1, 32),
        a['Wv'].T, a['bv'].reshape(1, 32), a['Wo'].T, a['bo'].reshape(1, 32),
        params['comb_W'], params['comb_b'].reshape(1, 32),
        params['rec_ro_W'], params['rec_ro_b'].reshape(1, 1),
        params['lig_ro_W'], params['lig_ro_b'].reshape(1, 1),
    )

    m = params['mlp']
    return _run_mlp(feats.reshape(_G, 128),
                    m['W1'], m['b1'].reshape(1, 256),
                    m['W2'], m['b2'].reshape(1, 128),
                    m['Wo'], m['bo'].reshape(1, 1))


# speedup vs baseline: 2.3154x; 1.6789x over previous
import jax
import jax.numpy as jnp
from jax import lax
from jax.experimental import pallas as pl
from jax.experimental.pallas import tpu as pltpu
from jax.experimental.pallas import tpu_sc as plsc

_G, _NPG, _N, _E = 10, 1000, 10000, 160000
_DH = 32
_NC, _NS = 2, 16
_NW = _NC * _NS
_RPT = 128
_NE2 = 2 * _E
_NROWS = _NE2 // _RPT
_N2 = 2 * _N


def _proj_body(lx_ref, feat_ref, emb_ref, lw_ref, lb_ref, rw_ref, rb_ref, out_ref):
    h_lig = jax.nn.relu(
        jnp.dot(lx_ref[...], lw_ref[...], preferred_element_type=jnp.float32)
        + lb_ref[...])
    emb_proj = jnp.dot(emb_ref[...], rw_ref[...], preferred_element_type=jnp.float32)
    feat = feat_ref[...]
    onehot = (lax.broadcasted_iota(jnp.int32, (feat.shape[0], 32), 1)
              == feat).astype(jnp.float32)
    h_rec = jax.nn.relu(
        jnp.dot(onehot, emb_proj, preferred_element_type=jnp.float32) + rb_ref[...])
    out_ref[0] = h_lig
    out_ref[1] = h_rec


def _run_proj(lig_x, feat, emb_pad, lw, lb, rw, rb):
    blk = 2000
    nb = _N // blk
    return pl.pallas_call(
        _proj_body,
        grid=(nb,),
        in_specs=[
            pl.BlockSpec((blk, 128), lambda i: (i, 0)),
            pl.BlockSpec((blk, 1), lambda i: (i, 0)),
            pl.BlockSpec((32, 64), lambda i: (0, 0)),
            pl.BlockSpec((128, 32), lambda i: (0, 0)),
            pl.BlockSpec((1, 32), lambda i: (0, 0)),
            pl.BlockSpec((64, 32), lambda i: (0, 0)),
            pl.BlockSpec((1, 32), lambda i: (0, 0)),
        ],
        out_specs=pl.BlockSpec((2, blk, 32), lambda i: (0, i, 0)),
        out_shape=jax.ShapeDtypeStruct((2, _N, 32), jnp.float32),
    )(lig_x, feat, emb_pad, lw, lb, rw, rb)


def _gather_body(tbl_hbm, idx_hbm, out_hbm, idx_v, rows_v, sem):
    wid = lax.axis_index("s") * _NC + lax.axis_index("c")
    n_extra = _NROWS - (_NROWS // _NW) * _NW
    n_it = jnp.where(wid < n_extra, _NROWS // _NW + 1, _NROWS // _NW)

    def step(t, _):
        j = wid + t * _NW
        pltpu.sync_copy(idx_hbm.at[j], idx_v)
        pltpu.async_copy(tbl_hbm.at[idx_v], rows_v, sem).wait()
        pltpu.sync_copy(rows_v, out_hbm.at[pl.ds(j * _RPT, _RPT)])
        return _

    lax.fori_loop(0, n_it, step, 0)


def _run_gather(tbl, idx2d):
    mesh = plsc.VectorSubcoreMesh(core_axis_name="c", subcore_axis_name="s")
    f = pl.kernel(
        _gather_body,
        out_type=jax.ShapeDtypeStruct((_NE2, 32), jnp.float32),
        mesh=mesh,
        scratch_types=[
            pltpu.VMEM((_RPT,), jnp.int32),
            pltpu.VMEM((_RPT, 32), jnp.float32),
            pltpu.SemaphoreType.DMA,
        ],
        compiler_params=pltpu.CompilerParams(use_tc_tiling_on_sc=False),
    )
    return f(tbl, idx2d)


def _msg_body(ea_ref, hs_ref, w1_ref, b1_ref, w2_ref, b2_ref, t_ref, s_ref,
              out_ref):
    u = jax.nn.relu(
        jnp.dot(ea_ref[0], w1_ref[0], preferred_element_type=jnp.float32)
        + b1_ref[0])
    w = jnp.dot(u.astype(jnp.bfloat16), w2_ref[0],
                preferred_element_type=jnp.float32) + b2_ref[0]
    hsb = jnp.dot(hs_ref[0], t_ref[...], preferred_element_type=jnp.float32)
    out_ref[0] = jnp.dot(w * hsb, s_ref[...],
                         preferred_element_type=jnp.float32)


def _run_msg(ea_s, hs_s, w1_s, b1_s, w2_s, b2_s, t0, s0):
    blk = 2000
    nb = _E // blk
    return pl.pallas_call(
        _msg_body,
        grid=(2, nb),
        in_specs=[
            pl.BlockSpec((1, blk, 16), lambda g, e: (g, e, 0)),
            pl.BlockSpec((1, blk, 32), lambda g, e: (g, e, 0)),
            pl.BlockSpec((1, 16, 128), lambda g, e: (g, 0, 0)),
            pl.BlockSpec((1, 1, 128), lambda g, e: (g, 0, 0)),
            pl.BlockSpec((1, 128, 1024), lambda g, e: (g, 0, 0)),
            pl.BlockSpec((1, 1, 1024), lambda g, e: (g, 0, 0)),
            pl.BlockSpec((32, 1024), lambda g, e: (0, 0)),
            pl.BlockSpec((1024, 32), lambda g, e: (0, 0)),
        ],
        compiler_params=pltpu.CompilerParams(
            dimension_semantics=("arbitrary", "arbitrary")),
        out_specs=pl.BlockSpec((1, blk, 32), lambda g, e: (g, e, 0)),
        out_shape=jax.ShapeDtypeStruct((2, _E, 32), jnp.float32),
    )(ea_s, hs_s, w1_s, b1_s, w2_s, b2_s, t0, s0)


def _scatter_body(msg_hbm, dst_hbm, zero_hbm, out_hbm, acc_sh, idx_v, rows_v):
    c = lax.axis_index("c")
    s = lax.axis_index("s")
    wid = s * _NC + c

    @pl.when(s == 0)
    def _():
        pltpu.sync_copy(zero_hbm, acc_sh)

    plsc.subcore_barrier()

    n_extra = _NROWS - (_NROWS // _NW) * _NW
    n_it = jnp.where(wid < n_extra, _NROWS // _NW + 1, _NROWS // _NW)

    def step(t, _):
        j = wid + t * _NW
        pltpu.sync_copy(dst_hbm.at[j], idx_v)
        pltpu.sync_copy(msg_hbm.at[pl.ds(j * _RPT, _RPT)], rows_v)
        pltpu.sync_copy(rows_v, acc_sh.at[idx_v], add=True)
        return _

    lax.fori_loop(0, n_it, step, 0)
    plsc.subcore_barrier()

    rows = _N2 // _NS
    pltpu.sync_copy(acc_sh.at[pl.ds(s * rows, rows)],
                    out_hbm.at[c, pl.ds(s * rows, rows)])


def _run_scatter(msg_cat, dst2d, zeros):
    mesh = plsc.VectorSubcoreMesh(core_axis_name="c", subcore_axis_name="s")
    f = pl.kernel(
        _scatter_body,
        out_type=jax.ShapeDtypeStruct((_NC, _N2, 32), jnp.float32),
        mesh=mesh,
        scratch_types=[
            pltpu.VMEM_SHARED((_N2, 32), jnp.float32),
            pltpu.VMEM((_RPT,), jnp.int32),
            pltpu.VMEM((_RPT, 32), jnp.float32),
        ],
        compiler_params=pltpu.CompilerParams(use_tc_tiling_on_sc=False),
    )
    return f(msg_cat, dst2d, zeros)


def _gru_body(parts_ref, nnb_ref, h_ref, wi_ref, bi_ref, wh_ref, bh_ref, out_ref):
    agg = parts_ref[0, 0] + parts_ref[1, 0] + nnb_ref[0]
    m = jax.nn.relu(agg)
    h = h_ref[0]
    gi = jnp.dot(m, wi_ref[0], preferred_element_type=jnp.float32) + bi_ref[0]
    gh = jnp.dot(h, wh_ref[0], preferred_element_type=jnp.float32) + bh_ref[0]
    r = jax.nn.sigmoid(gi[:, 0:32] + gh[:, 0:32])
    z = jax.nn.sigmoid(gi[:, 32:64] + gh[:, 32:64])
    n = jnp.tanh(gi[:, 64:96] + r * gh[:, 64:96])
    out_ref[0] = (1.0 - z) * n + z * h


def _run_gru(parts, nnb_s, h_s, wi_s, bi_s, wh_s, bh_s):
    blk = 2000
    nb = _N // blk
    return pl.pallas_call(
        _gru_body,
        grid=(2, nb),
        in_specs=[
            pl.BlockSpec((2, 1, blk, 32), lambda g, b: (0, g, b, 0)),
            pl.BlockSpec((1, 1, 32), lambda g, b: (g, 0, 0)),
            pl.BlockSpec((1, blk, 32), lambda g, b: (g, b, 0)),
            pl.BlockSpec((1, 32, 96), lambda g, b: (g, 0, 0)),
            pl.BlockSpec((1, 1, 96), lambda g, b: (g, 0, 0)),
            pl.BlockSpec((1, 32, 96), lambda g, b: (g, 0, 0)),
            pl.BlockSpec((1, 1, 96), lambda g, b: (g, 0, 0)),
        ],
        out_specs=pl.BlockSpec((1, blk, 32), lambda g, b: (g, b, 0)),
        out_shape=jax.ShapeDtypeStruct((2, _N, 32), jnp.float32),
    )(parts, nnb_s, h_s, wi_s, bi_s, wh_s, bh_s)


def _atn_body(hid_ref, wq_ref, bq_ref, wk_ref, bk_ref, wv_ref, bv_ref,
              wo_ref, bo_ref, cw_ref, cb_ref, rw_ref, rb_ref, lw_ref, lb_ref,
              out_ref):
    lig = hid_ref[0]
    rec = hid_ref[1]
    q = jnp.dot(lig, wq_ref[...], preferred_element_type=jnp.float32) + bq_ref[...]
    k = jnp.dot(rec, wk_ref[...], preferred_element_type=jnp.float32) + bk_ref[...]
    v = jnp.dot(rec, wv_ref[...], preferred_element_type=jnp.float32) + bv_ref[...]
    scores = lax.dot_general(q, k, (((1,), (1,)), ((), ())),
                             preferred_element_type=jnp.float32) * (1.0 / (_DH ** 0.5))
    mx = jnp.max(scores, axis=1, keepdims=True)
    ex = jnp.exp(scores - mx)
    a = ex / jnp.sum(ex, axis=1, keepdims=True)
    av = jnp.dot(a, v, preferred_element_type=jnp.float32)
    atn = jnp.dot(av, wo_ref[...], preferred_element_type=jnp.float32) + bo_ref[...]
    cat = jnp.concatenate([lig, atn], axis=1)
    lcomb = jnp.dot(cat, cw_ref[...], preferred_element_type=jnp.float32) + cb_ref[...]
    wr = jax.nn.sigmoid(
        jnp.dot(rec, rw_ref[...], preferred_element_type=jnp.float32) + rb_ref[...])
    hs_rec = jnp.sum(wr * rec, axis=0, keepdims=True)
    hm_rec = jnp.max(rec, axis=0, keepdims=True)
    wl = jax.nn.sigmoid(
        jnp.dot(lcomb, lw_ref[...], preferred_element_type=jnp.float32) + lb_ref[...])
    hs_lig = jnp.sum(wl * lcomb, axis=0, keepdims=True)
    hm_lig = jnp.max(lcomb, axis=0, keepdims=True)
    out_ref[0] = jnp.concatenate([hs_rec, hm_rec, hs_lig, hm_lig], axis=1)


def _run_atn(hid_s, wqt, bq, wkt, bk, wvt, bv, wot, bo, cw, cb, rw, rb, lw, lb):
    small = lambda shape: pl.BlockSpec(shape, lambda g: tuple(0 for _ in shape))
    return pl.pallas_call(
        _atn_body,
        grid=(_G,),
        in_specs=[
            pl.BlockSpec((2, _NPG, 32), lambda g: (0, g, 0)),
            small((32, 32)), small((1, 32)),
            small((32, 32)), small((1, 32)),
            small((32, 32)), small((1, 32)),
            small((32, 32)), small((1, 32)),
            small((64, 32)), small((1, 32)),
            small((32, 1)), small((1, 1)),
            small((32, 1)), small((1, 1)),
        ],
        out_specs=pl.BlockSpec((1, 1, 128), lambda g: (g, 0, 0)),
        out_shape=jax.ShapeDtypeStruct((_G, 1, 128), jnp.float32),
    )(hid_s, wqt, bq, wkt, bk, wvt, bv, wot, bo, cw, cb, rw, rb, lw, lb)


def _mlp_body(x_ref, w1_ref, b1_ref, w2_ref, b2_ref, wo_ref, bo_ref, out_ref):
    x = jnp.dot(x_ref[...], w1_ref[...], preferred_element_type=jnp.float32) + b1_ref[...]
    x = jnp.where(x > 0, x, 0.01 * x)
    x = jnp.dot(x, w2_ref[...], preferred_element_type=jnp.float32) + b2_ref[...]
    x = jnp.where(x > 0, x, 0.01 * x)
    out_ref[...] = jnp.dot(x, wo_ref[...], preferred_element_type=jnp.float32) + bo_ref[...]


def _run_mlp(x, w1, b1, w2, b2, wo, bo):
    return pl.pallas_call(
        _mlp_body,
        out_shape=jax.ShapeDtypeStruct((_G, 1), jnp.float32),
    )(x, w1, b1, w2, b2, wo, bo)


def kernel(lig_x, lig_edge_index, lig_edge_attr, rec_feat, rec_edge_index,
           rec_edge_attr, params):
    pg, pr = params['lig_gnn'], params['rec_gnn']
    f32 = jnp.float32

    emb_pad = jnp.zeros((32, 64), f32).at[:21].set(params['rec_embed'])
    h_s = _run_proj(lig_x, rec_feat, emb_pad,
                    pg['proj_W'], pg['proj_b'].reshape(1, 32),
                    pr['proj_W'], pr['proj_b'].reshape(1, 32))

    tbl = h_s.reshape(_N2, 32)
    src_cat = jnp.concatenate(
        [lig_edge_index[0], rec_edge_index[0] + _N]).reshape(_NROWS, _RPT)
    dst_cat = jnp.concatenate(
        [lig_edge_index[1], rec_edge_index[1] + _N]).reshape(_NROWS, _RPT)

    h_src = _run_gather(tbl, src_cat).reshape(2, _E, 32)

    ea_s = jnp.stack([lig_edge_attr, rec_edge_attr])
    w1_s = jnp.stack([pg['eW1'], pr['eW1']])
    b1_s = jnp.stack([pg['eb1'], pr['eb1']]).reshape(2, 1, 128)
    w2_s = jnp.stack([pg['eW2'], pr['eW2']]).astype(jnp.bfloat16)
    b2_s = jnp.stack([pg['eb2'], pr['eb2']]).reshape(2, 1, 1024)
    eye32 = jnp.eye(32, dtype=f32)
    t0 = jnp.repeat(eye32, 32, axis=1)
    s0 = jnp.tile(eye32, (32, 1))
    msg = _run_msg(ea_s, h_src, w1_s, b1_s, w2_s, b2_s, t0, s0)

    parts = _run_scatter(msg.reshape(_NE2, 32), dst_cat,
                         jnp.zeros((_N2, 32), f32)).reshape(2, 2, _N, 32)

    nnb_s = jnp.stack([pg['nn_bias'], pr['nn_bias']]).reshape(2, 1, 32)
    wi_s = jnp.stack([pg['gru_Wi'], pr['gru_Wi']])
    bi_s = jnp.stack([pg['gru_bi'], pr['gru_bi']]).reshape(2, 1, 96)
    wh_s = jnp.stack([pg['gru_Wh'], pr['gru_Wh']])
    bh_s = jnp.stack([pg['gru_bh'], pr['gru_bh']]).reshape(2, 1, 96)
    hid_s = _run_gru(parts, nnb_s, h_s, wi_s, bi_s, wh_s, bh_s)

    a = params['atn']
    feats = _run_atn(
        hid_s,
        a['Wq'].T, a['bq'].reshape(1, 32), a['Wk'].T, a['bk'].reshape(1, 32),
        a['Wv'].T, a['bv'].reshape(1, 32), a['Wo'].T, a['bo'].reshape(1, 32),
        params['comb_W'], params['comb_b'].reshape(1, 32),
        params['rec_ro_W'], params['rec_ro_b'].reshape(1, 1),
        params['lig_ro_W'], params['lig_ro_b'].reshape(1, 1),
    )

    m = params['mlp']
    return _run_mlp(feats.reshape(_G, 128),
                    m['W1'], m['b1'].reshape(1, 256),
                    m['W2'], m['b2'].reshape(1, 128),
                    m['Wo'], m['bo'].reshape(1, 1))


# speedup vs baseline: 2.8377x; 1.2256x over previous
import jax
import jax.numpy as jnp
from jax import lax
from jax.experimental import pallas as pl
from jax.experimental.pallas import tpu as pltpu
from jax.experimental.pallas import tpu_sc as plsc

_G, _NPG, _N, _E = 10, 1000, 10000, 160000
_DH = 32
_NC, _NS = 2, 16
_NW = _NC * _NS
_RPT = 128
_NE2 = 2 * _E
_NROWS = _NE2 // _RPT
_N2 = 2 * _N


def _proj_body(lx_ref, feat_ref, emb_ref, lw_ref, lb_ref, rw_ref, rb_ref, out_ref):
    h_lig = jax.nn.relu(
        jnp.dot(lx_ref[...], lw_ref[...], preferred_element_type=jnp.float32)
        + lb_ref[...])
    emb_proj = jnp.dot(emb_ref[...], rw_ref[...], preferred_element_type=jnp.float32)
    feat = feat_ref[...]
    onehot = (lax.broadcasted_iota(jnp.int32, (feat.shape[0], 32), 1)
              == feat).astype(jnp.float32)
    h_rec = jax.nn.relu(
        jnp.dot(onehot, emb_proj, preferred_element_type=jnp.float32) + rb_ref[...])
    out_ref[0] = h_lig
    out_ref[1] = h_rec


def _run_proj(lig_x, feat, emb_pad, lw, lb, rw, rb):
    blk = 2000
    nb = _N // blk
    return pl.pallas_call(
        _proj_body,
        grid=(nb,),
        in_specs=[
            pl.BlockSpec((blk, 128), lambda i: (i, 0)),
            pl.BlockSpec((blk, 1), lambda i: (i, 0)),
            pl.BlockSpec((32, 64), lambda i: (0, 0)),
            pl.BlockSpec((128, 32), lambda i: (0, 0)),
            pl.BlockSpec((1, 32), lambda i: (0, 0)),
            pl.BlockSpec((64, 32), lambda i: (0, 0)),
            pl.BlockSpec((1, 32), lambda i: (0, 0)),
        ],
        out_specs=pl.BlockSpec((2, blk, 32), lambda i: (0, i, 0)),
        out_shape=jax.ShapeDtypeStruct((2, _N, 32), jnp.float32),
    )(lig_x, feat, emb_pad, lw, lb, rw, rb)


def _gather_body(tbl_hbm, idx_hbm, out_hbm, idx_v, rows_v, sem):
    wid = lax.axis_index("s") * _NC + lax.axis_index("c")
    n_extra = _NROWS - (_NROWS // _NW) * _NW
    n_it = jnp.where(wid < n_extra, _NROWS // _NW + 1, _NROWS // _NW)

    def step(t, _):
        j = wid + t * _NW
        pltpu.sync_copy(idx_hbm.at[j], idx_v)
        pltpu.async_copy(tbl_hbm.at[idx_v], rows_v, sem).wait()
        pltpu.sync_copy(rows_v, out_hbm.at[pl.ds(j * _RPT, _RPT)])
        return _

    lax.fori_loop(0, n_it, step, 0)


def _run_gather(tbl, idx2d):
    mesh = plsc.VectorSubcoreMesh(core_axis_name="c", subcore_axis_name="s")
    f = pl.kernel(
        _gather_body,
        out_type=jax.ShapeDtypeStruct((_NE2, 32), jnp.float32),
        mesh=mesh,
        scratch_types=[
            pltpu.VMEM((_RPT,), jnp.int32),
            pltpu.VMEM((_RPT, 32), jnp.float32),
            pltpu.SemaphoreType.DMA,
        ],
        compiler_params=pltpu.CompilerParams(use_tc_tiling_on_sc=False),
    )
    return f(tbl, idx2d)


def _msg_body(ea_ref, hs_ref, w1_ref, b1_ref, w2_ref, b2_ref, t_ref, out_ref):
    u = jax.nn.relu(
        jnp.dot(ea_ref[0], w1_ref[0], preferred_element_type=jnp.float32)
        + b1_ref[0])
    w = jnp.dot(u.astype(jnp.bfloat16), w2_ref[0],
                preferred_element_type=jnp.float32) + b2_ref[0]
    hsb = jnp.dot(hs_ref[0], t_ref[...], preferred_element_type=jnp.float32)
    p = w * hsb
    q = p[:, 0:512] + p[:, 512:1024]
    q = q[:, 0:256] + q[:, 256:512]
    q = q[:, 0:128] + q[:, 128:256]
    out_ref[0] = (q[:, 0:32] + q[:, 32:64]) + (q[:, 64:96] + q[:, 96:128])


def _run_msg(ea_s, hs_s, w1_s, b1_s, w2_s, b2_s, t0):
    blk = 2000
    nb = _E // blk
    return pl.pallas_call(
        _msg_body,
        grid=(2, nb),
        in_specs=[
            pl.BlockSpec((1, blk, 16), lambda g, e: (g, e, 0)),
            pl.BlockSpec((1, blk, 32), lambda g, e: (g, e, 0)),
            pl.BlockSpec((1, 16, 128), lambda g, e: (g, 0, 0)),
            pl.BlockSpec((1, 1, 128), lambda g, e: (g, 0, 0)),
            pl.BlockSpec((1, 128, 1024), lambda g, e: (g, 0, 0)),
            pl.BlockSpec((1, 1, 1024), lambda g, e: (g, 0, 0)),
            pl.BlockSpec((32, 1024), lambda g, e: (0, 0)),
        ],
        compiler_params=pltpu.CompilerParams(
            dimension_semantics=("arbitrary", "arbitrary")),
        out_specs=pl.BlockSpec((1, blk, 32), lambda g, e: (g, e, 0)),
        out_shape=jax.ShapeDtypeStruct((2, _E, 32), jnp.float32),
    )(ea_s, hs_s, w1_s, b1_s, w2_s, b2_s, t0)


def _scatter_body(msg_hbm, dst_hbm, zero_hbm, out_hbm, acc_sh, idx_v, rows_v):
    c = lax.axis_index("c")
    s = lax.axis_index("s")
    wid = s * _NC + c

    @pl.when(s == 0)
    def _():
        pltpu.sync_copy(zero_hbm, acc_sh)

    plsc.subcore_barrier()

    n_extra = _NROWS - (_NROWS // _NW) * _NW
    n_it = jnp.where(wid < n_extra, _NROWS // _NW + 1, _NROWS // _NW)

    def step(t, _):
        j = wid + t * _NW
        pltpu.sync_copy(dst_hbm.at[j], idx_v)
        pltpu.sync_copy(msg_hbm.at[pl.ds(j * _RPT, _RPT)], rows_v)
        pltpu.sync_copy(rows_v, acc_sh.at[idx_v], add=True)
        return _

    lax.fori_loop(0, n_it, step, 0)
    plsc.subcore_barrier()

    rows = _N2 // _NS
    pltpu.sync_copy(acc_sh.at[pl.ds(s * rows, rows)],
                    out_hbm.at[c, pl.ds(s * rows, rows)])


def _run_scatter(msg_cat, dst2d, zeros):
    mesh = plsc.VectorSubcoreMesh(core_axis_name="c", subcore_axis_name="s")
    f = pl.kernel(
        _scatter_body,
        out_type=jax.ShapeDtypeStruct((_NC, _N2, 32), jnp.float32),
        mesh=mesh,
        scratch_types=[
            pltpu.VMEM_SHARED((_N2, 32), jnp.float32),
            pltpu.VMEM((_RPT,), jnp.int32),
            pltpu.VMEM((_RPT, 32), jnp.float32),
        ],
        compiler_params=pltpu.CompilerParams(use_tc_tiling_on_sc=False),
    )
    return f(msg_cat, dst2d, zeros)


def _gru_body(parts_ref, nnb_ref, h_ref, wi_ref, bi_ref, wh_ref, bh_ref, out_ref):
    agg = parts_ref[0, 0] + parts_ref[1, 0] + nnb_ref[0]
    m = jax.nn.relu(agg)
    h = h_ref[0]
    gi = jnp.dot(m, wi_ref[0], preferred_element_type=jnp.float32) + bi_ref[0]
    gh = jnp.dot(h, wh_ref[0], preferred_element_type=jnp.float32) + bh_ref[0]
    r = jax.nn.sigmoid(gi[:, 0:32] + gh[:, 0:32])
    z = jax.nn.sigmoid(gi[:, 32:64] + gh[:, 32:64])
    n = jnp.tanh(gi[:, 64:96] + r * gh[:, 64:96])
    out_ref[0] = (1.0 - z) * n + z * h


def _run_gru(parts, nnb_s, h_s, wi_s, bi_s, wh_s, bh_s):
    blk = 2000
    nb = _N // blk
    return pl.pallas_call(
        _gru_body,
        grid=(2, nb),
        in_specs=[
            pl.BlockSpec((2, 1, blk, 32), lambda g, b: (0, g, b, 0)),
            pl.BlockSpec((1, 1, 32), lambda g, b: (g, 0, 0)),
            pl.BlockSpec((1, blk, 32), lambda g, b: (g, b, 0)),
            pl.BlockSpec((1, 32, 96), lambda g, b: (g, 0, 0)),
            pl.BlockSpec((1, 1, 96), lambda g, b: (g, 0, 0)),
            pl.BlockSpec((1, 32, 96), lambda g, b: (g, 0, 0)),
            pl.BlockSpec((1, 1, 96), lambda g, b: (g, 0, 0)),
        ],
        out_specs=pl.BlockSpec((1, blk, 32), lambda g, b: (g, b, 0)),
        out_shape=jax.ShapeDtypeStruct((2, _N, 32), jnp.float32),
    )(parts, nnb_s, h_s, wi_s, bi_s, wh_s, bh_s)


def _atn_body(hid_ref, wq_ref, bq_ref, wk_ref, bk_ref, wv_ref, bv_ref,
              wo_ref, bo_ref, cw_ref, cb_ref, rw_ref, rb_ref, lw_ref, lb_ref,
              out_ref):
    lig = hid_ref[0]
    rec = hid_ref[1]
    q = jnp.dot(lig, wq_ref[...], preferred_element_type=jnp.float32) + bq_ref[...]
    k = jnp.dot(rec, wk_ref[...], preferred_element_type=jnp.float32) + bk_ref[...]
    v = jnp.dot(rec, wv_ref[...], preferred_element_type=jnp.float32) + bv_ref[...]
    scores = lax.dot_general(q, k, (((1,), (1,)), ((), ())),
                             preferred_element_type=jnp.float32) * (1.0 / (_DH ** 0.5))
    mx = jnp.max(scores, axis=1, keepdims=True)
    ex = jnp.exp(scores - mx)
    a = ex / jnp.sum(ex, axis=1, keepdims=True)
    av = jnp.dot(a, v, preferred_element_type=jnp.float32)
    atn = jnp.dot(av, wo_ref[...], preferred_element_type=jnp.float32) + bo_ref[...]
    cat = jnp.concatenate([lig, atn], axis=1)
    lcomb = jnp.dot(cat, cw_ref[...], preferred_element_type=jnp.float32) + cb_ref[...]
    wr = jax.nn.sigmoid(
        jnp.dot(rec, rw_ref[...], preferred_element_type=jnp.float32) + rb_ref[...])
    hs_rec = jnp.sum(wr * rec, axis=0, keepdims=True)
    hm_rec = jnp.max(rec, axis=0, keepdims=True)
    wl = jax.nn.sigmoid(
        jnp.dot(lcomb, lw_ref[...], preferred_element_type=jnp.float32) + lb_ref[...])
    hs_lig = jnp.sum(wl * lcomb, axis=0, keepdims=True)
    hm_lig = jnp.max(lcomb, axis=0, keepdims=True)
    out_ref[0] = jnp.concatenate([hs_rec, hm_rec, hs_lig, hm_lig], axis=1)


def _run_atn(hid_s, wqt, bq, wkt, bk, wvt, bv, wot, bo, cw, cb, rw, rb, lw, lb):
    small = lambda shape: pl.BlockSpec(shape, lambda g: tuple(0 for _ in shape))
    return pl.pallas_call(
        _atn_body,
        grid=(_G,),
        in_specs=[
            pl.BlockSpec((2, _NPG, 32), lambda g: (0, g, 0)),
            small((32, 32)), small((1, 32)),
            small((32, 32)), small((1, 32)),
            small((32, 32)), small((1, 32)),
            small((32, 32)), small((1, 32)),
            small((64, 32)), small((1, 32)),
            small((32, 1)), small((1, 1)),
            small((32, 1)), small((1, 1)),
        ],
        out_specs=pl.BlockSpec((1, 1, 128), lambda g: (g, 0, 0)),
        out_shape=jax.ShapeDtypeStruct((_G, 1, 128), jnp.float32),
    )(hid_s, wqt, bq, wkt, bk, wvt, bv, wot, bo, cw, cb, rw, rb, lw, lb)


def _mlp_body(x_ref, w1_ref, b1_ref, w2_ref, b2_ref, wo_ref, bo_ref, out_ref):
    x = jnp.dot(x_ref[...], w1_ref[...], preferred_element_type=jnp.float32) + b1_ref[...]
    x = jnp.where(x > 0, x, 0.01 * x)
    x = jnp.dot(x, w2_ref[...], preferred_element_type=jnp.float32) + b2_ref[...]
    x = jnp.where(x > 0, x, 0.01 * x)
    out_ref[...] = jnp.dot(x, wo_ref[...], preferred_element_type=jnp.float32) + bo_ref[...]


def _run_mlp(x, w1, b1, w2, b2, wo, bo):
    return pl.pallas_call(
        _mlp_body,
        out_shape=jax.ShapeDtypeStruct((_G, 1), jnp.float32),
    )(x, w1, b1, w2, b2, wo, bo)


def kernel(lig_x, lig_edge_index, lig_edge_attr, rec_feat, rec_edge_index,
           rec_edge_attr, params):
    pg, pr = params['lig_gnn'], params['rec_gnn']
    f32 = jnp.float32

    emb_pad = jnp.zeros((32, 64), f32).at[:21].set(params['rec_embed'])
    h_s = _run_proj(lig_x, rec_feat, emb_pad,
                    pg['proj_W'], pg['proj_b'].reshape(1, 32),
                    pr['proj_W'], pr['proj_b'].reshape(1, 32))

    tbl = h_s.reshape(_N2, 32)
    src_cat = jnp.concatenate(
        [lig_edge_index[0], rec_edge_index[0] + _N]).reshape(_NROWS, _RPT)
    dst_cat = jnp.concatenate(
        [lig_edge_index[1], rec_edge_index[1] + _N]).reshape(_NROWS, _RPT)

    h_src = _run_gather(tbl, src_cat).reshape(2, _E, 32)

    ea_s = jnp.stack([lig_edge_attr, rec_edge_attr])
    w1_s = jnp.stack([pg['eW1'], pr['eW1']])
    b1_s = jnp.stack([pg['eb1'], pr['eb1']]).reshape(2, 1, 128)
    w2_s = jnp.stack([pg['eW2'], pr['eW2']]).astype(jnp.bfloat16)
    b2_s = jnp.stack([pg['eb2'], pr['eb2']]).reshape(2, 1, 1024)
    t0 = jnp.repeat(jnp.eye(32, dtype=f32), 32, axis=1)
    msg = _run_msg(ea_s, h_src, w1_s, b1_s, w2_s, b2_s, t0)

    parts = _run_scatter(msg.reshape(_NE2, 32), dst_cat,
                         jnp.zeros((_N2, 32), f32)).reshape(2, 2, _N, 32)

    nnb_s = jnp.stack([pg['nn_bias'], pr['nn_bias']]).reshape(2, 1, 32)
    wi_s = jnp.stack([pg['gru_Wi'], pr['gru_Wi']])
    bi_s = jnp.stack([pg['gru_bi'], pr['gru_bi']]).reshape(2, 1, 96)
    wh_s = jnp.stack([pg['gru_Wh'], pr['gru_Wh']])
    bh_s = jnp.stack([pg['gru_bh'], pr['gru_bh']]).reshape(2, 1, 96)
    hid_s = _run_gru(parts, nnb_s, h_s, wi_s, bi_s, wh_s, bh_s)

    a = params['atn']
    feats = _run_atn(
        hid_s,
        a['Wq'].T, a['bq'].reshape(1, 32), a['Wk'].T, a['bk'].reshape(1, 32),
        a['Wv'].T, a['bv'].reshape(1, 32), a['Wo'].T, a['bo'].reshape(1, 32),
        params['comb_W'], params['comb_b'].reshape(1, 32),
        params['rec_ro_W'], params['rec_ro_b'].reshape(1, 1),
        params['lig_ro_W'], params['lig_ro_b'].reshape(1, 1),
    )

    m = params['mlp']
    return _run_mlp(feats.reshape(_G, 128),
                    m['W1'], m['b1'].reshape(1, 256),
                    m['W2'], m['b2'].reshape(1, 128),
                    m['Wo'], m['bo'].reshape(1, 1))


# speedup vs baseline: 3.1303x; 1.1031x over previous
import jax
import jax.numpy as jnp
from jax import lax
from jax.experimental import pallas as pl
from jax.experimental.pallas import tpu as pltpu
from jax.experimental.pallas import tpu_sc as plsc

_G, _NPG, _N, _E = 10, 1000, 10000, 160000
_DH = 32
_NC, _NS = 2, 16
_NW = _NC * _NS
_RPT = 128
_ER = _E // _RPT
_ER_BASE = _ER // _NW
_ER_EXTRA = _ER - _ER_BASE * _NW


def _proj_body(lx_ref, feat_ref, emb_ref, lw_ref, lb_ref, rw_ref, rb_ref,
               lout_ref, rout_ref):
    lout_ref[...] = jax.nn.relu(
        jnp.dot(lx_ref[...], lw_ref[...], preferred_element_type=jnp.float32)
        + lb_ref[...])
    emb_proj = jnp.dot(emb_ref[...], rw_ref[...], preferred_element_type=jnp.float32)
    feat = feat_ref[...]
    onehot = (lax.broadcasted_iota(jnp.int32, (feat.shape[0], 32), 1)
              == feat).astype(jnp.float32)
    rout_ref[...] = jax.nn.relu(
        jnp.dot(onehot, emb_proj, preferred_element_type=jnp.float32) + rb_ref[...])


def _run_proj(lig_x, feat, emb_pad, lw, lb, rw, rb):
    blk = 2000
    nb = _N // blk
    return pl.pallas_call(
        _proj_body,
        grid=(nb,),
        in_specs=[
            pl.BlockSpec((blk, 128), lambda i: (i, 0)),
            pl.BlockSpec((blk, 1), lambda i: (i, 0)),
            pl.BlockSpec((32, 64), lambda i: (0, 0)),
            pl.BlockSpec((128, 32), lambda i: (0, 0)),
            pl.BlockSpec((1, 32), lambda i: (0, 0)),
            pl.BlockSpec((64, 32), lambda i: (0, 0)),
            pl.BlockSpec((1, 32), lambda i: (0, 0)),
        ],
        out_specs=[pl.BlockSpec((blk, 32), lambda i: (i, 0)),
                   pl.BlockSpec((blk, 32), lambda i: (i, 0))],
        out_shape=[jax.ShapeDtypeStruct((_N, 32), jnp.float32),
                   jax.ShapeDtypeStruct((_N, 32), jnp.float32)],
    )(lig_x, feat, emb_pad, lw, lb, rw, rb)


def _gather_body(tbl_hbm, idx_hbm, out_hbm, idx_v, rows_v, sem):
    wid = lax.axis_index("s") * _NC + lax.axis_index("c")
    n_it = jnp.where(wid < _ER_EXTRA, _ER_BASE + 1, _ER_BASE)

    def step(t, _):
        j = wid + t * _NW
        pltpu.sync_copy(idx_hbm.at[j], idx_v)
        pltpu.async_copy(tbl_hbm.at[idx_v], rows_v, sem).wait()
        pltpu.sync_copy(rows_v, out_hbm.at[pl.ds(j * _RPT, _RPT)])
        return _

    lax.fori_loop(0, n_it, step, 0)


def _run_gather(tbl, idx2d):
    mesh = plsc.VectorSubcoreMesh(core_axis_name="c", subcore_axis_name="s")
    f = pl.kernel(
        _gather_body,
        out_type=jax.ShapeDtypeStruct((_E, 32), jnp.float32),
        mesh=mesh,
        scratch_types=[
            pltpu.VMEM((_RPT,), jnp.int32),
            pltpu.VMEM((_RPT, 32), jnp.float32),
            pltpu.SemaphoreType.DMA,
        ],
        compiler_params=pltpu.CompilerParams(use_tc_tiling_on_sc=False),
    )
    return f(tbl, idx2d)


def _msg_body(ea_ref, hs_ref, w1_ref, b1_ref, w2_ref, b2_ref, t_ref, out_ref):
    u = jax.nn.relu(
        jnp.dot(ea_ref[...], w1_ref[...], preferred_element_type=jnp.float32)
        + b1_ref[...])
    w = jnp.dot(u.astype(jnp.bfloat16), w2_ref[...],
                preferred_element_type=jnp.float32) + b2_ref[...]
    hsb = jnp.dot(hs_ref[...], t_ref[...], preferred_element_type=jnp.float32)
    p = w * hsb
    q = p[:, 0:512] + p[:, 512:1024]
    q = q[:, 0:256] + q[:, 256:512]
    q = q[:, 0:128] + q[:, 128:256]
    out_ref[...] = (q[:, 0:32] + q[:, 32:64]) + (q[:, 64:96] + q[:, 96:128])


def _run_msg(ea, hs, w1, b1, w2, b2, t0):
    blk = 2000
    nb = _E // blk
    return pl.pallas_call(
        _msg_body,
        grid=(nb,),
        in_specs=[
            pl.BlockSpec((blk, 16), lambda e: (e, 0)),
            pl.BlockSpec((blk, 32), lambda e: (e, 0)),
            pl.BlockSpec((16, 128), lambda e: (0, 0)),
            pl.BlockSpec((1, 128), lambda e: (0, 0)),
            pl.BlockSpec((128, 1024), lambda e: (0, 0)),
            pl.BlockSpec((1, 1024), lambda e: (0, 0)),
            pl.BlockSpec((32, 1024), lambda e: (0, 0)),
        ],
        out_specs=pl.BlockSpec((blk, 32), lambda e: (e, 0)),
        out_shape=jax.ShapeDtypeStruct((_E, 32), jnp.float32),
    )(ea, hs, w1, b1, w2, b2, t0)


def _scatter_body(msg_hbm, dst_hbm, zero_hbm, out_hbm, acc_sh, idx_v, rows_v):
    c = lax.axis_index("c")
    s = lax.axis_index("s")
    wid = s * _NC + c

    @pl.when(s == 0)
    def _():
        pltpu.sync_copy(zero_hbm, acc_sh)

    plsc.subcore_barrier()

    n_it = jnp.where(wid < _ER_EXTRA, _ER_BASE + 1, _ER_BASE)

    def step(t, _):
        j = wid + t * _NW
        pltpu.sync_copy(dst_hbm.at[j], idx_v)
        pltpu.sync_copy(msg_hbm.at[pl.ds(j * _RPT, _RPT)], rows_v)
        pltpu.sync_copy(rows_v, acc_sh.at[idx_v], add=True)
        return _

    lax.fori_loop(0, n_it, step, 0)
    plsc.subcore_barrier()

    rows = _N // _NS
    pltpu.sync_copy(acc_sh.at[pl.ds(s * rows, rows)],
                    out_hbm.at[c, pl.ds(s * rows, rows)])


def _run_scatter(msg, dst2d, zeros):
    mesh = plsc.VectorSubcoreMesh(core_axis_name="c", subcore_axis_name="s")
    f = pl.kernel(
        _scatter_body,
        out_type=jax.ShapeDtypeStruct((_NC, _N, 32), jnp.float32),
        mesh=mesh,
        scratch_types=[
            pltpu.VMEM_SHARED((_N, 32), jnp.float32),
            pltpu.VMEM((_RPT,), jnp.int32),
            pltpu.VMEM((_RPT, 32), jnp.float32),
        ],
        compiler_params=pltpu.CompilerParams(use_tc_tiling_on_sc=False),
    )
    return f(msg, dst2d, zeros)


def _gru_body(parts_ref, nnb_ref, h_ref, wi_ref, bi_ref, wh_ref, bh_ref, out_ref):
    agg = parts_ref[0] + parts_ref[1] + nnb_ref[...]
    m = jax.nn.relu(agg)
    h = h_ref[...]
    gi = jnp.dot(m, wi_ref[...], preferred_element_type=jnp.float32) + bi_ref[...]
    gh = jnp.dot(h, wh_ref[...], preferred_element_type=jnp.float32) + bh_ref[...]
    r = jax.nn.sigmoid(gi[:, 0:32] + gh[:, 0:32])
    z = jax.nn.sigmoid(gi[:, 32:64] + gh[:, 32:64])
    n = jnp.tanh(gi[:, 64:96] + r * gh[:, 64:96])
    out_ref[...] = (1.0 - z) * n + z * h


def _run_gru(parts, nnb, h, wi, bi, wh, bh):
    blk = 2000
    nb = _N // blk
    return pl.pallas_call(
        _gru_body,
        grid=(nb,),
        in_specs=[
            pl.BlockSpec((2, blk, 32), lambda b: (0, b, 0)),
            pl.BlockSpec((1, 32), lambda b: (0, 0)),
            pl.BlockSpec((blk, 32), lambda b: (b, 0)),
            pl.BlockSpec((32, 96), lambda b: (0, 0)),
            pl.BlockSpec((1, 96), lambda b: (0, 0)),
            pl.BlockSpec((32, 96), lambda b: (0, 0)),
            pl.BlockSpec((1, 96), lambda b: (0, 0)),
        ],
        out_specs=pl.BlockSpec((blk, 32), lambda b: (b, 0)),
        out_shape=jax.ShapeDtypeStruct((_N, 32), jnp.float32),
    )(parts, nnb, h, wi, bi, wh, bh)


def _atn_body(lig_ref, rec_ref, wq_ref, bq_ref, wk_ref, bk_ref, wv_ref, bv_ref,
              wo_ref, bo_ref, cw_ref, cb_ref, rw_ref, rb_ref, lw_ref, lb_ref,
              out_ref):
    lig = lig_ref[...]
    rec = rec_ref[...]
    q = jnp.dot(lig, wq_ref[...], preferred_element_type=jnp.float32) + bq_ref[...]
    k = jnp.dot(rec, wk_ref[...], preferred_element_type=jnp.float32) + bk_ref[...]
    v = jnp.dot(rec, wv_ref[...], preferred_element_type=jnp.float32) + bv_ref[...]
    scores = lax.dot_general(q, k, (((1,), (1,)), ((), ())),
                             preferred_element_type=jnp.float32) * (1.0 / (_DH ** 0.5))
    mx = jnp.max(scores, axis=1, keepdims=True)
    ex = jnp.exp(scores - mx)
    a = ex / jnp.sum(ex, axis=1, keepdims=True)
    av = jnp.dot(a, v, preferred_element_type=jnp.float32)
    atn = jnp.dot(av, wo_ref[...], preferred_element_type=jnp.float32) + bo_ref[...]
    cat = jnp.concatenate([lig, atn], axis=1)
    lcomb = jnp.dot(cat, cw_ref[...], preferred_element_type=jnp.float32) + cb_ref[...]
    wr = jax.nn.sigmoid(
        jnp.dot(rec, rw_ref[...], preferred_element_type=jnp.float32) + rb_ref[...])
    hs_rec = jnp.sum(wr * rec, axis=0, keepdims=True)
    hm_rec = jnp.max(rec, axis=0, keepdims=True)
    wl = jax.nn.sigmoid(
        jnp.dot(lcomb, lw_ref[...], preferred_element_type=jnp.float32) + lb_ref[...])
    hs_lig = jnp.sum(wl * lcomb, axis=0, keepdims=True)
    hm_lig = jnp.max(lcomb, axis=0, keepdims=True)
    out_ref[0] = jnp.concatenate([hs_rec, hm_rec, hs_lig, hm_lig], axis=1)


def _run_atn(hid_lig, hid_rec, wqt, bq, wkt, bk, wvt, bv, wot, bo, cw, cb,
             rw, rb, lw, lb):
    small = lambda shape: pl.BlockSpec(shape, lambda g: tuple(0 for _ in shape))
    return pl.pallas_call(
        _atn_body,
        grid=(_G,),
        in_specs=[
            pl.BlockSpec((_NPG, 32), lambda g: (g, 0)),
            pl.BlockSpec((_NPG, 32), lambda g: (g, 0)),
            small((32, 32)), small((1, 32)),
            small((32, 32)), small((1, 32)),
            small((32, 32)), small((1, 32)),
            small((32, 32)), small((1, 32)),
            small((64, 32)), small((1, 32)),
            small((32, 1)), small((1, 1)),
            small((32, 1)), small((1, 1)),
        ],
        out_specs=pl.BlockSpec((1, 1, 128), lambda g: (g, 0, 0)),
        out_shape=jax.ShapeDtypeStruct((_G, 1, 128), jnp.float32),
    )(hid_lig, hid_rec, wqt, bq, wkt, bk, wvt, bv, wot, bo, cw, cb, rw, rb, lw, lb)


def _mlp_body(x_ref, w1_ref, b1_ref, w2_ref, b2_ref, wo_ref, bo_ref, out_ref):
    x = jnp.dot(x_ref[...], w1_ref[...], preferred_element_type=jnp.float32) + b1_ref[...]
    x = jnp.where(x > 0, x, 0.01 * x)
    x = jnp.dot(x, w2_ref[...], preferred_element_type=jnp.float32) + b2_ref[...]
    x = jnp.where(x > 0, x, 0.01 * x)
    out_ref[...] = jnp.dot(x, wo_ref[...], preferred_element_type=jnp.float32) + bo_ref[...]


def _run_mlp(x, w1, b1, w2, b2, wo, bo):
    return pl.pallas_call(
        _mlp_body,
        out_shape=jax.ShapeDtypeStruct((_G, 1), jnp.float32),
    )(x, w1, b1, w2, b2, wo, bo)


def kernel(lig_x, lig_edge_index, lig_edge_attr, rec_feat, rec_edge_index,
           rec_edge_attr, params):
    pg, pr = params['lig_gnn'], params['rec_gnn']
    f32 = jnp.float32

    emb_pad = jnp.zeros((32, 64), f32).at[:21].set(params['rec_embed'])
    h_lig, h_rec = _run_proj(lig_x, rec_feat, emb_pad,
                             pg['proj_W'], pg['proj_b'].reshape(1, 32),
                             pr['proj_W'], pr['proj_b'].reshape(1, 32))

    src_lig = lig_edge_index[0].reshape(_ER, _RPT)
    dst_lig = lig_edge_index[1].reshape(_ER, _RPT)
    src_rec = rec_edge_index[0].reshape(_ER, _RPT)
    dst_rec = rec_edge_index[1].reshape(_ER, _RPT)

    hsrc_lig = _run_gather(h_lig, src_lig)
    hsrc_rec = _run_gather(h_rec, src_rec)

    t0 = jnp.repeat(jnp.eye(32, dtype=f32), 32, axis=1)
    msg_lig = _run_msg(lig_edge_attr, hsrc_lig,
                       pg['eW1'], pg['eb1'].reshape(1, 128),
                       pg['eW2'].astype(jnp.bfloat16),
                       pg['eb2'].reshape(1, 1024), t0)
    msg_rec = _run_msg(rec_edge_attr, hsrc_rec,
                       pr['eW1'], pr['eb1'].reshape(1, 128),
                       pr['eW2'].astype(jnp.bfloat16),
                       pr['eb2'].reshape(1, 1024), t0)

    zeros = jnp.zeros((_N, 32), f32)
    parts_lig = _run_scatter(msg_lig, dst_lig, zeros)
    parts_rec = _run_scatter(msg_rec, dst_rec, zeros)

    hid_lig = _run_gru(parts_lig, pg['nn_bias'].reshape(1, 32), h_lig,
                       pg['gru_Wi'], pg['gru_bi'].reshape(1, 96),
                       pg['gru_Wh'], pg['gru_bh'].reshape(1, 96))
    hid_rec = _run_gru(parts_rec, pr['nn_bias'].reshape(1, 32), h_rec,
                       pr['gru_Wi'], pr['gru_bi'].reshape(1, 96),
                       pr['gru_Wh'], pr['gru_bh'].reshape(1, 96))

    a = params['atn']
    feats = _run_atn(
        hid_lig, hid_rec,
        a['Wq'].T, a['bq'].reshape(1, 32), a['Wk'].T, a['bk'].reshape(1, 32),
        a['Wv'].T, a['bv'].reshape(1, 32), a['Wo'].T, a['bo'].reshape(1, 32),
        params['comb_W'], params['comb_b'].reshape(1, 32),
        params['rec_ro_W'], params['rec_ro_b'].reshape(1, 1),
        params['lig_ro_W'], params['lig_ro_b'].reshape(1, 1),
    )

    m = params['mlp']
    return _run_mlp(feats.reshape(_G, 128),
                    m['W1'], m['b1'].reshape(1, 256),
                    m['W2'], m['b2'].reshape(1, 128),
                    m['Wo'], m['bo'].reshape(1, 1))


# speedup vs baseline: 3.1400x; 1.0031x over previous
import jax
import jax.numpy as jnp
from jax import lax
from jax.experimental import pallas as pl
from jax.experimental.pallas import tpu as pltpu
from jax.experimental.pallas import tpu_sc as plsc

_G, _NPG, _N, _E = 10, 1000, 10000, 160000
_DH = 32
_NC, _NS = 2, 16
_NW = _NC * _NS
_RPT = 128
_ER = _E // _RPT
_ER_BASE = _ER // _NW
_ER_EXTRA = _ER - _ER_BASE * _NW


def _proj_body(lx_ref, feat_ref, emb_ref, lw_ref, lb_ref, rw_ref, rb_ref,
               lout_ref, rout_ref):
    lout_ref[...] = jax.nn.relu(
        jnp.dot(lx_ref[...], lw_ref[...], preferred_element_type=jnp.float32)
        + lb_ref[...])
    emb_proj = jnp.dot(emb_ref[...], rw_ref[...], preferred_element_type=jnp.float32)
    feat = feat_ref[...]
    onehot = (lax.broadcasted_iota(jnp.int32, (feat.shape[0], 32), 1)
              == feat).astype(jnp.float32)
    rout_ref[...] = jax.nn.relu(
        jnp.dot(onehot, emb_proj, preferred_element_type=jnp.float32) + rb_ref[...])


def _run_proj(lig_x, feat, emb_pad, lw, lb, rw, rb):
    blk = 2000
    nb = _N // blk
    return pl.pallas_call(
        _proj_body,
        grid=(nb,),
        in_specs=[
            pl.BlockSpec((blk, 128), lambda i: (i, 0)),
            pl.BlockSpec((blk, 1), lambda i: (i, 0)),
            pl.BlockSpec((32, 64), lambda i: (0, 0)),
            pl.BlockSpec((128, 32), lambda i: (0, 0)),
            pl.BlockSpec((1, 32), lambda i: (0, 0)),
            pl.BlockSpec((64, 32), lambda i: (0, 0)),
            pl.BlockSpec((1, 32), lambda i: (0, 0)),
        ],
        out_specs=[pl.BlockSpec((blk, 32), lambda i: (i, 0)),
                   pl.BlockSpec((blk, 32), lambda i: (i, 0))],
        out_shape=[jax.ShapeDtypeStruct((_N, 32), jnp.float32),
                   jax.ShapeDtypeStruct((_N, 32), jnp.float32)],
    )(lig_x, feat, emb_pad, lw, lb, rw, rb)


def _gather_body(tbl_hbm, idx_hbm, out_hbm, idx_v, rows_v, sem):
    wid = lax.axis_index("s") * _NC + lax.axis_index("c")
    n_it = jnp.where(wid < _ER_EXTRA, _ER_BASE + 1, _ER_BASE)

    def step(t, _):
        j = wid + t * _NW
        pltpu.sync_copy(idx_hbm.at[j], idx_v)
        pltpu.async_copy(tbl_hbm.at[idx_v], rows_v, sem).wait()
        pltpu.sync_copy(rows_v, out_hbm.at[pl.ds(j * _RPT, _RPT)])
        return _

    lax.fori_loop(0, n_it, step, 0)


def _run_gather(tbl, idx2d):
    mesh = plsc.VectorSubcoreMesh(core_axis_name="c", subcore_axis_name="s")
    f = pl.kernel(
        _gather_body,
        out_type=jax.ShapeDtypeStruct((_E, 32), jnp.float32),
        mesh=mesh,
        scratch_types=[
            pltpu.VMEM((_RPT,), jnp.int32),
            pltpu.VMEM((_RPT, 32), jnp.float32),
            pltpu.SemaphoreType.DMA,
        ],
        compiler_params=pltpu.CompilerParams(use_tc_tiling_on_sc=False),
    )
    return f(tbl, idx2d)


def _msg_body(ea_ref, hs_ref, w1_ref, b1_ref, w2_ref, b2_ref, t_ref, out_ref):
    u = jax.nn.relu(
        jnp.dot(ea_ref[...], w1_ref[...], preferred_element_type=jnp.float32)
        + b1_ref[...])
    w = jnp.dot(u, w2_ref[...], preferred_element_type=jnp.float32) + b2_ref[...]
    hsb = jnp.dot(hs_ref[...], t_ref[...], preferred_element_type=jnp.float32)
    p = w * hsb
    q = p[:, 0:512] + p[:, 512:1024]
    q = q[:, 0:256] + q[:, 256:512]
    q = q[:, 0:128] + q[:, 128:256]
    out_ref[...] = (q[:, 0:32] + q[:, 32:64]) + (q[:, 64:96] + q[:, 96:128])


def _run_msg(ea, hs, w1, b1, w2, b2, t0):
    blk = 2000
    nb = _E // blk
    return pl.pallas_call(
        _msg_body,
        grid=(nb,),
        in_specs=[
            pl.BlockSpec((blk, 16), lambda e: (e, 0)),
            pl.BlockSpec((blk, 32), lambda e: (e, 0)),
            pl.BlockSpec((16, 128), lambda e: (0, 0)),
            pl.BlockSpec((1, 128), lambda e: (0, 0)),
            pl.BlockSpec((128, 1024), lambda e: (0, 0)),
            pl.BlockSpec((1, 1024), lambda e: (0, 0)),
            pl.BlockSpec((32, 1024), lambda e: (0, 0)),
        ],
        out_specs=pl.BlockSpec((blk, 32), lambda e: (e, 0)),
        out_shape=jax.ShapeDtypeStruct((_E, 32), jnp.float32),
    )(ea, hs, w1, b1, w2, b2, t0)


def _scatter_body(msg_hbm, dst_hbm, zero_hbm, out_hbm, acc_sh, idx_v, rows_v):
    c = lax.axis_index("c")
    s = lax.axis_index("s")
    wid = s * _NC + c

    @pl.when(s == 0)
    def _():
        pltpu.sync_copy(zero_hbm, acc_sh)

    plsc.subcore_barrier()

    n_it = jnp.where(wid < _ER_EXTRA, _ER_BASE + 1, _ER_BASE)

    def step(t, _):
        j = wid + t * _NW
        pltpu.sync_copy(dst_hbm.at[j], idx_v)
        pltpu.sync_copy(msg_hbm.at[pl.ds(j * _RPT, _RPT)], rows_v)
        pltpu.sync_copy(rows_v, acc_sh.at[idx_v], add=True)
        return _

    lax.fori_loop(0, n_it, step, 0)
    plsc.subcore_barrier()

    rows = _N // _NS
    pltpu.sync_copy(acc_sh.at[pl.ds(s * rows, rows)],
                    out_hbm.at[c, pl.ds(s * rows, rows)])


def _run_scatter(msg, dst2d, zeros):
    mesh = plsc.VectorSubcoreMesh(core_axis_name="c", subcore_axis_name="s")
    f = pl.kernel(
        _scatter_body,
        out_type=jax.ShapeDtypeStruct((_NC, _N, 32), jnp.float32),
        mesh=mesh,
        scratch_types=[
            pltpu.VMEM_SHARED((_N, 32), jnp.float32),
            pltpu.VMEM((_RPT,), jnp.int32),
            pltpu.VMEM((_RPT, 32), jnp.float32),
        ],
        compiler_params=pltpu.CompilerParams(use_tc_tiling_on_sc=False),
    )
    return f(msg, dst2d, zeros)


def _gru_body(parts_ref, nnb_ref, h_ref, wi_ref, bi_ref, wh_ref, bh_ref, out_ref):
    agg = parts_ref[0] + parts_ref[1] + nnb_ref[...]
    m = jax.nn.relu(agg)
    h = h_ref[...]
    gi = jnp.dot(m, wi_ref[...], preferred_element_type=jnp.float32) + bi_ref[...]
    gh = jnp.dot(h, wh_ref[...], preferred_element_type=jnp.float32) + bh_ref[...]
    r = jax.nn.sigmoid(gi[:, 0:32] + gh[:, 0:32])
    z = jax.nn.sigmoid(gi[:, 32:64] + gh[:, 32:64])
    n = jnp.tanh(gi[:, 64:96] + r * gh[:, 64:96])
    out_ref[...] = (1.0 - z) * n + z * h


def _run_gru(parts, nnb, h, wi, bi, wh, bh):
    blk = 2000
    nb = _N // blk
    return pl.pallas_call(
        _gru_body,
        grid=(nb,),
        in_specs=[
            pl.BlockSpec((2, blk, 32), lambda b: (0, b, 0)),
            pl.BlockSpec((1, 32), lambda b: (0, 0)),
            pl.BlockSpec((blk, 32), lambda b: (b, 0)),
            pl.BlockSpec((32, 96), lambda b: (0, 0)),
            pl.BlockSpec((1, 96), lambda b: (0, 0)),
            pl.BlockSpec((32, 96), lambda b: (0, 0)),
            pl.BlockSpec((1, 96), lambda b: (0, 0)),
        ],
        out_specs=pl.BlockSpec((blk, 32), lambda b: (b, 0)),
        out_shape=jax.ShapeDtypeStruct((_N, 32), jnp.float32),
    )(parts, nnb, h, wi, bi, wh, bh)


def _atn_body(lig_ref, rec_ref, wq_ref, bq_ref, wk_ref, bk_ref, wv_ref, bv_ref,
              wo_ref, bo_ref, cw_ref, cb_ref, rw_ref, rb_ref, lw_ref, lb_ref,
              out_ref):
    lig = lig_ref[...]
    rec = rec_ref[...]
    q = jnp.dot(lig, wq_ref[...], preferred_element_type=jnp.float32) + bq_ref[...]
    k = jnp.dot(rec, wk_ref[...], preferred_element_type=jnp.float32) + bk_ref[...]
    v = jnp.dot(rec, wv_ref[...], preferred_element_type=jnp.float32) + bv_ref[...]
    scores = lax.dot_general(q, k, (((1,), (1,)), ((), ())),
                             preferred_element_type=jnp.float32) * (1.0 / (_DH ** 0.5))
    mx = jnp.max(scores, axis=1, keepdims=True)
    ex = jnp.exp(scores - mx)
    a = ex / jnp.sum(ex, axis=1, keepdims=True)
    av = jnp.dot(a, v, preferred_element_type=jnp.float32)
    atn = jnp.dot(av, wo_ref[...], preferred_element_type=jnp.float32) + bo_ref[...]
    cat = jnp.concatenate([lig, atn], axis=1)
    lcomb = jnp.dot(cat, cw_ref[...], preferred_element_type=jnp.float32) + cb_ref[...]
    wr = jax.nn.sigmoid(
        jnp.dot(rec, rw_ref[...], preferred_element_type=jnp.float32) + rb_ref[...])
    hs_rec = jnp.sum(wr * rec, axis=0, keepdims=True)
    hm_rec = jnp.max(rec, axis=0, keepdims=True)
    wl = jax.nn.sigmoid(
        jnp.dot(lcomb, lw_ref[...], preferred_element_type=jnp.float32) + lb_ref[...])
    hs_lig = jnp.sum(wl * lcomb, axis=0, keepdims=True)
    hm_lig = jnp.max(lcomb, axis=0, keepdims=True)
    out_ref[0] = jnp.concatenate([hs_rec, hm_rec, hs_lig, hm_lig], axis=1)


def _run_atn(hid_lig, hid_rec, wqt, bq, wkt, bk, wvt, bv, wot, bo, cw, cb,
             rw, rb, lw, lb):
    small = lambda shape: pl.BlockSpec(shape, lambda g: tuple(0 for _ in shape))
    return pl.pallas_call(
        _atn_body,
        grid=(_G,),
        in_specs=[
            pl.BlockSpec((_NPG, 32), lambda g: (g, 0)),
            pl.BlockSpec((_NPG, 32), lambda g: (g, 0)),
            small((32, 32)), small((1, 32)),
            small((32, 32)), small((1, 32)),
            small((32, 32)), small((1, 32)),
            small((32, 32)), small((1, 32)),
            small((64, 32)), small((1, 32)),
            small((32, 1)), small((1, 1)),
            small((32, 1)), small((1, 1)),
        ],
        out_specs=pl.BlockSpec((1, 1, 128), lambda g: (g, 0, 0)),
        out_shape=jax.ShapeDtypeStruct((_G, 1, 128), jnp.float32),
    )(hid_lig, hid_rec, wqt, bq, wkt, bk, wvt, bv, wot, bo, cw, cb, rw, rb, lw, lb)


def _mlp_body(x_ref, w1_ref, b1_ref, w2_ref, b2_ref, wo_ref, bo_ref, out_ref):
    x = jnp.dot(x_ref[...], w1_ref[...], preferred_element_type=jnp.float32) + b1_ref[...]
    x = jnp.where(x > 0, x, 0.01 * x)
    x = jnp.dot(x, w2_ref[...], preferred_element_type=jnp.float32) + b2_ref[...]
    x = jnp.where(x > 0, x, 0.01 * x)
    out_ref[...] = jnp.dot(x, wo_ref[...], preferred_element_type=jnp.float32) + bo_ref[...]


def _run_mlp(x, w1, b1, w2, b2, wo, bo):
    return pl.pallas_call(
        _mlp_body,
        out_shape=jax.ShapeDtypeStruct((_G, 1), jnp.float32),
    )(x, w1, b1, w2, b2, wo, bo)


def kernel(lig_x, lig_edge_index, lig_edge_attr, rec_feat, rec_edge_index,
           rec_edge_attr, params):
    pg, pr = params['lig_gnn'], params['rec_gnn']
    f32 = jnp.float32

    emb_pad = jnp.zeros((32, 64), f32).at[:21].set(params['rec_embed'])
    h_lig, h_rec = _run_proj(lig_x, rec_feat, emb_pad,
                             pg['proj_W'], pg['proj_b'].reshape(1, 32),
                             pr['proj_W'], pr['proj_b'].reshape(1, 32))

    src_lig = lig_edge_index[0].reshape(_ER, _RPT)
    dst_lig = lig_edge_index[1].reshape(_ER, _RPT)
    src_rec = rec_edge_index[0].reshape(_ER, _RPT)
    dst_rec = rec_edge_index[1].reshape(_ER, _RPT)

    hsrc_lig = _run_gather(h_lig, src_lig)
    hsrc_rec = _run_gather(h_rec, src_rec)

    t0 = jnp.repeat(jnp.eye(32, dtype=f32), 32, axis=1)
    msg_lig = _run_msg(lig_edge_attr, hsrc_lig,
                       pg['eW1'], pg['eb1'].reshape(1, 128),
                       pg['eW2'], pg['eb2'].reshape(1, 1024), t0)
    msg_rec = _run_msg(rec_edge_attr, hsrc_rec,
                       pr['eW1'], pr['eb1'].reshape(1, 128),
                       pr['eW2'], pr['eb2'].reshape(1, 1024), t0)

    zeros = jnp.zeros((_N, 32), f32)
    parts_lig = _run_scatter(msg_lig, dst_lig, zeros)
    parts_rec = _run_scatter(msg_rec, dst_rec, zeros)

    hid_lig = _run_gru(parts_lig, pg['nn_bias'].reshape(1, 32), h_lig,
                       pg['gru_Wi'], pg['gru_bi'].reshape(1, 96),
                       pg['gru_Wh'], pg['gru_bh'].reshape(1, 96))
    hid_rec = _run_gru(parts_rec, pr['nn_bias'].reshape(1, 32), h_rec,
                       pr['gru_Wi'], pr['gru_bi'].reshape(1, 96),
                       pr['gru_Wh'], pr['gru_bh'].reshape(1, 96))

    a = params['atn']
    feats = _run_atn(
        hid_lig, hid_rec,
        a['Wq'].T, a['bq'].reshape(1, 32), a['Wk'].T, a['bk'].reshape(1, 32),
        a['Wv'].T, a['bv'].reshape(1, 32), a['Wo'].T, a['bo'].reshape(1, 32),
        params['comb_W'], params['comb_b'].reshape(1, 32),
        params['rec_ro_W'], params['rec_ro_b'].reshape(1, 1),
        params['lig_ro_W'], params['lig_ro_b'].reshape(1, 1),
    )

    m = params['mlp']
    return _run_mlp(feats.reshape(_G, 128),
                    m['W1'], m['b1'].reshape(1, 256),
                    m['W2'], m['b2'].reshape(1, 128),
                    m['Wo'], m['bo'].reshape(1, 1))
